# Initial kernel scaffold; baseline (speedup 1.0000x reference)
import jax
import jax.numpy as jnp
from jax.experimental import pallas as pl


def kernel(x, pos_emb):
    raise NotImplementedError("write your pallas kernel here")


# speedup vs baseline: 3.1781x; 3.1781x over previous
import jax
import jax.numpy as jnp
from jax.experimental import pallas as pl


_SCALE = 32.0
_BS = 1024


def _body(x_ref, pe_ref, o_ref):
    o_ref[...] = x_ref[...] * _SCALE + pe_ref[...][None, :, :]


def kernel(x, pos_emb):
    b, s, d = x.shape
    n_s = s // _BS
    return pl.pallas_call(
        _body,
        grid=(n_s, b),
        in_specs=[
            pl.BlockSpec((1, _BS, d), lambda i, j: (j, i, 0)),
            pl.BlockSpec((_BS, d), lambda i, j: (i, 0)),
        ],
        out_specs=pl.BlockSpec((1, _BS, d), lambda i, j: (j, i, 0)),
        out_shape=jax.ShapeDtypeStruct((b, s, d), x.dtype),
    )(x, pos_emb[:s])


# speedup vs baseline: 3.3064x; 1.0404x over previous
import jax
import jax.numpy as jnp
from jax.experimental import pallas as pl


_SCALE = 32.0
_BS = 2048


def _body(x_ref, pe_ref, o_ref):
    o_ref[...] = x_ref[...] * _SCALE + pe_ref[...][None, :, :]


def kernel(x, pos_emb):
    b, s, d = x.shape
    n_s = s // _BS
    return pl.pallas_call(
        _body,
        grid=(n_s, b),
        in_specs=[
            pl.BlockSpec((1, _BS, d), lambda i, j: (j, i, 0)),
            pl.BlockSpec((_BS, d), lambda i, j: (i, 0)),
        ],
        out_specs=pl.BlockSpec((1, _BS, d), lambda i, j: (j, i, 0)),
        out_shape=jax.ShapeDtypeStruct((b, s, d), x.dtype),
    )(x, pos_emb[:s])
